# baseline (speedup 1.0000x reference)
import functools

import jax
import jax.numpy as jnp
from jax import lax
from jax.experimental import pallas as pl
from jax.experimental.pallas import tpu as pltpu
from jax.experimental.pallas import tpu_sc as plsc

_NC = 2
_NS = 16
_NW = _NC * _NS
_LANES = 16

_TC_TILE = 512
_NCHUNKS = 2


def _argmax_body(trt_ref, keyst_ref, idx_ref):
    tr_cm = trt_ref[0]
    keys_cm = keyst_ref[...]
    s = lax.dot_general(tr_cm, keys_cm, (((0,), (0,)), ((), ())),
                        preferred_element_type=jnp.float32)
    idx_ref[0, 0, :] = jnp.argmax(s, axis=1).astype(jnp.int32)


def _tc_argmax(trt3, keys_t, row_off, n_rows):
    b, c, l = trt3.shape
    m = keys_t.shape[1]
    jpr = l // _TC_TILE
    chunk_t = n_rows * l
    idx3 = pl.pallas_call(
        _argmax_body,
        grid=(n_rows, jpr),
        in_specs=[
            pl.BlockSpec((1, c, _TC_TILE), lambda bb, j: (bb + row_off, 0, j)),
            pl.BlockSpec((c, m), lambda bb, j: (0, 0)),
        ],
        out_specs=pl.BlockSpec((1, 1, _TC_TILE),
                               lambda bb, j: (bb * jpr + j, 0, 0)),
        out_shape=jax.ShapeDtypeStruct((n_rows * jpr, 1, _TC_TILE), jnp.int32),
        compiler_params=pltpu.CompilerParams(
            dimension_semantics=("arbitrary", "arbitrary")),
    )(trt3, keys_t)
    return idx3.reshape(chunk_t)


def _make_sc_mse(b, l, c, m, chunk_t):
    per_w = chunk_t // _NW
    n_groups = per_w // _LANES
    wpr = l // per_w
    mesh = plsc.VectorSubcoreMesh(core_axis_name="c", subcore_axis_name="s",
                                  num_cores=_NC, num_subcores=_NS)

    def make(row_off):
        @functools.partial(
            pl.kernel,
            out_type=[jax.ShapeDtypeStruct((chunk_t,), jnp.float32),
                      jax.ShapeDtypeStruct((chunk_t,), jnp.float32)],
            mesh=mesh,
            scratch_types=[
                pltpu.VMEM((c, m), jnp.float32),
                pltpu.VMEM((c, m), jnp.float32),
                pltpu.VMEM((c, per_w), jnp.float32),
                pltpu.VMEM((c, per_w), jnp.float32),
                pltpu.VMEM((per_w,), jnp.int32),
                pltpu.VMEM((per_w,), jnp.float32),
                pltpu.VMEM((per_w,), jnp.float32),
            ],
            compiler_params=pltpu.CompilerParams(needs_layout_passes=False),
        )
        def sc_mse(trt_hbm, rept_hbm, keyst_hbm, valuest_hbm, idx_hbm,
                   outk_hbm, outv_hbm,
                   keys_v, values_v, tr_v, rep_v, idx_v, outk_v, outv_v):
            wid = lax.axis_index("s") * _NC + lax.axis_index("c")
            base = wid * per_w
            bb = row_off + wid // wpr
            l0 = (wid % wpr) * per_w
            pltpu.sync_copy(keyst_hbm, keys_v)
            pltpu.sync_copy(valuest_hbm, values_v)
            pltpu.sync_copy(trt_hbm.at[bb, :, pl.ds(l0, per_w)], tr_v)
            pltpu.sync_copy(rept_hbm.at[bb, :, pl.ds(l0, per_w)], rep_v)
            pltpu.sync_copy(idx_hbm.at[pl.ds(base, per_w)], idx_v)
            lanes = lax.iota(jnp.int32, _LANES)

            def group_body(g, carry):
                rows = [g * 2 * _LANES + lanes, g * 2 * _LANES + _LANES + lanes]
                idxs = [plsc.load_gather(idx_v, [r]) for r in rows]
                acck = [[jnp.zeros((_LANES,), jnp.float32) for _ in range(2)]
                        for _ in range(2)]
                accv = [[jnp.zeros((_LANES,), jnp.float32) for _ in range(2)]
                        for _ in range(2)]
                for cc in range(c):
                    col = jnp.full((_LANES,), cc, jnp.int32)
                    par = cc & 1
                    for gi in range(2):
                        trc = plsc.load_gather(tr_v, [col, rows[gi]])
                        kc = plsc.load_gather(keys_v, [col, idxs[gi]])
                        dk = trc - kc
                        acck[gi][par] = acck[gi][par] + dk * dk
                        rc = plsc.load_gather(rep_v, [col, rows[gi]])
                        vc = plsc.load_gather(values_v, [col, idxs[gi]])
                        dv = rc - vc
                        accv[gi][par] = accv[gi][par] + dv * dv
                for gi in range(2):
                    plsc.store_scatter(outk_v, [rows[gi]],
                                       acck[gi][0] + acck[gi][1])
                    plsc.store_scatter(outv_v, [rows[gi]],
                                       accv[gi][0] + accv[gi][1])
                return carry

            lax.fori_loop(0, n_groups // 2, group_body, 0)
            pltpu.sync_copy(outk_v, outk_hbm.at[pl.ds(base, per_w)])
            pltpu.sync_copy(outv_v, outv_hbm.at[pl.ds(base, per_w)])

        return sc_mse

    return make


def kernel(trend_representation, representation, keys, values):
    b, l, c = trend_representation.shape
    m = keys.shape[0]
    trt3 = jnp.transpose(trend_representation, (0, 2, 1))
    rept3 = jnp.transpose(representation, (0, 2, 1))
    keys_t = keys.T
    values_t = values.T
    rows_per_chunk = b // _NCHUNKS
    chunk_t = rows_per_chunk * l
    sc_make = _make_sc_mse(b, l, c, m, chunk_t)
    kgs, vgs = [], []
    for i in range(_NCHUNKS):
        row_off = i * rows_per_chunk
        idx = _tc_argmax(trt3, keys_t, row_off, rows_per_chunk)
        kg, vg = sc_make(row_off)(trt3, rept3, keys_t, values_t, idx)
        kgs.append(kg)
        vgs.append(vg)
    kg = jnp.concatenate(kgs).reshape(b, l)
    vg = jnp.concatenate(vgs).reshape(b, l)
    return kg, vg


# speedup vs baseline: 1.0177x; 1.0177x over previous
import functools

import jax
import jax.numpy as jnp
from jax import lax
from jax.experimental import pallas as pl
from jax.experimental.pallas import tpu as pltpu
from jax.experimental.pallas import tpu_sc as plsc

_NC = 2
_NS = 16
_NW = _NC * _NS
_LANES = 16

_TC_TILE = 512
_NCHUNKS = 2


def _argmax_body(trt_ref, keyst_ref, idx_ref):
    tr_cm = trt_ref[0]
    keys_cm = keyst_ref[...]
    s = lax.dot_general(tr_cm, keys_cm, (((0,), (0,)), ((), ())),
                        preferred_element_type=jnp.float32)
    idx_ref[0, 0, :] = jnp.argmax(s, axis=1).astype(jnp.int32)


def _tc_argmax(trt3, keys_t, row_off, n_rows):
    b, c, l = trt3.shape
    m = keys_t.shape[1]
    jpr = l // _TC_TILE
    chunk_t = n_rows * l
    idx3 = pl.pallas_call(
        _argmax_body,
        grid=(n_rows, jpr),
        in_specs=[
            pl.BlockSpec((1, c, _TC_TILE), lambda bb, j: (bb + row_off, 0, j)),
            pl.BlockSpec((c, m), lambda bb, j: (0, 0)),
        ],
        out_specs=pl.BlockSpec((1, 1, _TC_TILE),
                               lambda bb, j: (bb * jpr + j, 0, 0)),
        out_shape=jax.ShapeDtypeStruct((n_rows * jpr, 1, _TC_TILE), jnp.int32),
        compiler_params=pltpu.CompilerParams(
            dimension_semantics=("arbitrary", "arbitrary"),
            fuse_transposed_lhs_in_matmul=True),
    )(trt3, keys_t)
    return idx3.reshape(chunk_t)


def _make_sc_mse(b, l, c, m, chunk_t):
    per_w = chunk_t // _NW
    n_groups = per_w // _LANES
    wpr = l // per_w
    mesh = plsc.VectorSubcoreMesh(core_axis_name="c", subcore_axis_name="s",
                                  num_cores=_NC, num_subcores=_NS)

    def make(row_off):
        @functools.partial(
            pl.kernel,
            out_type=[jax.ShapeDtypeStruct((chunk_t,), jnp.float32),
                      jax.ShapeDtypeStruct((chunk_t,), jnp.float32)],
            mesh=mesh,
            scratch_types=[
                pltpu.VMEM((c, m), jnp.float32),
                pltpu.VMEM((c, m), jnp.float32),
                pltpu.VMEM((c, per_w), jnp.float32),
                pltpu.VMEM((c, per_w), jnp.float32),
                pltpu.VMEM((per_w,), jnp.int32),
                pltpu.VMEM((per_w,), jnp.float32),
                pltpu.VMEM((per_w,), jnp.float32),
                pltpu.SemaphoreType.DMA,
                pltpu.SemaphoreType.DMA,
                pltpu.SemaphoreType.DMA,
                pltpu.SemaphoreType.DMA,
                pltpu.SemaphoreType.DMA,
            ],
            compiler_params=pltpu.CompilerParams(needs_layout_passes=False),
        )
        def sc_mse(trt_hbm, rept_hbm, keyst_hbm, valuest_hbm, idx_hbm,
                   outk_hbm, outv_hbm,
                   keys_v, values_v, tr_v, rep_v, idx_v, outk_v, outv_v,
                   sem0, sem1, sem2, sem3, sem4):
            wid = lax.axis_index("s") * _NC + lax.axis_index("c")
            base = wid * per_w
            bb = row_off + wid // wpr
            l0 = (wid % wpr) * per_w
            cps = [
                pltpu.async_copy(keyst_hbm, keys_v, sem0),
                pltpu.async_copy(valuest_hbm, values_v, sem1),
                pltpu.async_copy(trt_hbm.at[bb, :, pl.ds(l0, per_w)], tr_v,
                                 sem2),
                pltpu.async_copy(rept_hbm.at[bb, :, pl.ds(l0, per_w)], rep_v,
                                 sem3),
                pltpu.async_copy(idx_hbm.at[pl.ds(base, per_w)], idx_v, sem4),
            ]
            for cp in cps:
                cp.wait()
            lanes = lax.iota(jnp.int32, _LANES)

            def group_body(g, carry):
                rows = [g * 2 * _LANES + lanes, g * 2 * _LANES + _LANES + lanes]
                idxs = [plsc.load_gather(idx_v, [r]) for r in rows]
                acck = [[jnp.zeros((_LANES,), jnp.float32) for _ in range(2)]
                        for _ in range(2)]
                accv = [[jnp.zeros((_LANES,), jnp.float32) for _ in range(2)]
                        for _ in range(2)]
                for cc in range(c):
                    col = jnp.full((_LANES,), cc, jnp.int32)
                    par = cc & 1
                    for gi in range(2):
                        trc = plsc.load_gather(tr_v, [col, rows[gi]])
                        kc = plsc.load_gather(keys_v, [col, idxs[gi]])
                        dk = trc - kc
                        acck[gi][par] = acck[gi][par] + dk * dk
                        rc = plsc.load_gather(rep_v, [col, rows[gi]])
                        vc = plsc.load_gather(values_v, [col, idxs[gi]])
                        dv = rc - vc
                        accv[gi][par] = accv[gi][par] + dv * dv
                for gi in range(2):
                    plsc.store_scatter(outk_v, [rows[gi]],
                                       acck[gi][0] + acck[gi][1])
                    plsc.store_scatter(outv_v, [rows[gi]],
                                       accv[gi][0] + accv[gi][1])
                return carry

            lax.fori_loop(0, n_groups // 2, group_body, 0)
            pltpu.sync_copy(outk_v, outk_hbm.at[pl.ds(base, per_w)])
            pltpu.sync_copy(outv_v, outv_hbm.at[pl.ds(base, per_w)])

        return sc_mse

    return make


def kernel(trend_representation, representation, keys, values):
    b, l, c = trend_representation.shape
    m = keys.shape[0]
    trt3 = jnp.transpose(trend_representation, (0, 2, 1))
    rept3 = jnp.transpose(representation, (0, 2, 1))
    keys_t = keys.T
    values_t = values.T
    rows_per_chunk = b // _NCHUNKS
    chunk_t = rows_per_chunk * l
    sc_make = _make_sc_mse(b, l, c, m, chunk_t)
    kgs, vgs = [], []
    for i in range(_NCHUNKS):
        row_off = i * rows_per_chunk
        idx = _tc_argmax(trt3, keys_t, row_off, rows_per_chunk)
        kg, vg = sc_make(row_off)(trt3, rept3, keys_t, values_t, idx)
        kgs.append(kg)
        vgs.append(vg)
    kg = jnp.concatenate(kgs).reshape(b, l)
    vg = jnp.concatenate(vgs).reshape(b, l)
    return kg, vg


# speedup vs baseline: 1.1257x; 1.1061x over previous
import functools

import jax
import jax.numpy as jnp
from jax import lax
from jax.experimental import pallas as pl
from jax.experimental.pallas import tpu as pltpu
from jax.experimental.pallas import tpu_sc as plsc

_NC = 2
_NS = 16
_NW = _NC * _NS
_LANES = 16

_TC_TILE = 1024
_NCHUNKS = 2


def _argmax_body(trt_ref, keyst_ref, idx_ref):
    tr_cm = trt_ref[0]
    keys_cm = keyst_ref[...]
    s = lax.dot_general(tr_cm, keys_cm, (((0,), (0,)), ((), ())),
                        preferred_element_type=jnp.float32)
    idx_ref[0, 0, :] = jnp.argmax(s, axis=1).astype(jnp.int32)


def _tc_argmax(trt3, keys_t, row_off, n_rows):
    b, c, l = trt3.shape
    m = keys_t.shape[1]
    jpr = l // _TC_TILE
    chunk_t = n_rows * l
    idx3 = pl.pallas_call(
        _argmax_body,
        grid=(n_rows, jpr),
        in_specs=[
            pl.BlockSpec((1, c, _TC_TILE), lambda bb, j: (bb + row_off, 0, j)),
            pl.BlockSpec((c, m), lambda bb, j: (0, 0)),
        ],
        out_specs=pl.BlockSpec((1, 1, _TC_TILE),
                               lambda bb, j: (bb * jpr + j, 0, 0)),
        out_shape=jax.ShapeDtypeStruct((n_rows * jpr, 1, _TC_TILE), jnp.int32),
        compiler_params=pltpu.CompilerParams(
            dimension_semantics=("arbitrary", "arbitrary"),
            fuse_transposed_lhs_in_matmul=True),
    )(trt3, keys_t)
    return idx3.reshape(chunk_t)


def _make_sc_mse(b, l, c, m, chunk_t):
    per_w = chunk_t // _NW
    n_groups = per_w // _LANES
    wpr = l // per_w
    mesh = plsc.VectorSubcoreMesh(core_axis_name="c", subcore_axis_name="s",
                                  num_cores=_NC, num_subcores=_NS)

    def make(row_off):
        @functools.partial(
            pl.kernel,
            out_type=[jax.ShapeDtypeStruct((chunk_t,), jnp.float32),
                      jax.ShapeDtypeStruct((chunk_t,), jnp.float32)],
            mesh=mesh,
            scratch_types=[
                pltpu.VMEM((c, m), jnp.float32),
                pltpu.VMEM((c, m), jnp.float32),
                pltpu.VMEM((c, per_w), jnp.float32),
                pltpu.VMEM((c, per_w), jnp.float32),
                pltpu.VMEM((per_w,), jnp.int32),
                pltpu.VMEM((per_w,), jnp.float32),
                pltpu.VMEM((per_w,), jnp.float32),
                pltpu.SemaphoreType.DMA,
                pltpu.SemaphoreType.DMA,
                pltpu.SemaphoreType.DMA,
                pltpu.SemaphoreType.DMA,
                pltpu.SemaphoreType.DMA,
            ],
            compiler_params=pltpu.CompilerParams(needs_layout_passes=False),
        )
        def sc_mse(trt_hbm, rept_hbm, keyst_hbm, valuest_hbm, idx_hbm,
                   outk_hbm, outv_hbm,
                   keys_v, values_v, tr_v, rep_v, idx_v, outk_v, outv_v,
                   sem0, sem1, sem2, sem3, sem4):
            wid = lax.axis_index("s") * _NC + lax.axis_index("c")
            base = wid * per_w
            bb = row_off + wid // wpr
            l0 = (wid % wpr) * per_w
            cps = [
                pltpu.async_copy(keyst_hbm, keys_v, sem0),
                pltpu.async_copy(valuest_hbm, values_v, sem1),
                pltpu.async_copy(trt_hbm.at[bb, :, pl.ds(l0, per_w)], tr_v,
                                 sem2),
                pltpu.async_copy(rept_hbm.at[bb, :, pl.ds(l0, per_w)], rep_v,
                                 sem3),
                pltpu.async_copy(idx_hbm.at[pl.ds(base, per_w)], idx_v, sem4),
            ]
            for cp in cps:
                cp.wait()
            lanes = lax.iota(jnp.int32, _LANES)

            def group_body(g, carry):
                rows = [g * 2 * _LANES + lanes, g * 2 * _LANES + _LANES + lanes]
                idxs = [plsc.load_gather(idx_v, [r]) for r in rows]
                acck = [[jnp.zeros((_LANES,), jnp.float32) for _ in range(2)]
                        for _ in range(2)]
                accv = [[jnp.zeros((_LANES,), jnp.float32) for _ in range(2)]
                        for _ in range(2)]
                for cc in range(c):
                    col = jnp.full((_LANES,), cc, jnp.int32)
                    par = cc & 1
                    for gi in range(2):
                        trc = plsc.load_gather(tr_v, [col, rows[gi]])
                        kc = plsc.load_gather(keys_v, [col, idxs[gi]])
                        dk = trc - kc
                        acck[gi][par] = acck[gi][par] + dk * dk
                        rc = plsc.load_gather(rep_v, [col, rows[gi]])
                        vc = plsc.load_gather(values_v, [col, idxs[gi]])
                        dv = rc - vc
                        accv[gi][par] = accv[gi][par] + dv * dv
                for gi in range(2):
                    plsc.store_scatter(outk_v, [rows[gi]],
                                       acck[gi][0] + acck[gi][1])
                    plsc.store_scatter(outv_v, [rows[gi]],
                                       accv[gi][0] + accv[gi][1])
                return carry

            lax.fori_loop(0, n_groups // 2, group_body, 0)
            pltpu.sync_copy(outk_v, outk_hbm.at[pl.ds(base, per_w)])
            pltpu.sync_copy(outv_v, outv_hbm.at[pl.ds(base, per_w)])

        return sc_mse

    return make


def kernel(trend_representation, representation, keys, values):
    b, l, c = trend_representation.shape
    m = keys.shape[0]
    trt3 = jnp.transpose(trend_representation, (0, 2, 1))
    rept3 = jnp.transpose(representation, (0, 2, 1))
    keys_t = keys.T
    values_t = values.T
    rows_per_chunk = b // _NCHUNKS
    chunk_t = rows_per_chunk * l
    sc_make = _make_sc_mse(b, l, c, m, chunk_t)
    kgs, vgs = [], []
    for i in range(_NCHUNKS):
        row_off = i * rows_per_chunk
        idx = _tc_argmax(trt3, keys_t, row_off, rows_per_chunk)
        kg, vg = sc_make(row_off)(trt3, rept3, keys_t, values_t, idx)
        kgs.append(kg)
        vgs.append(vg)
    kg = jnp.concatenate(kgs).reshape(b, l)
    vg = jnp.concatenate(vgs).reshape(b, l)
    return kg, vg


# speedup vs baseline: 1.5037x; 1.3358x over previous
import functools

import jax
import jax.numpy as jnp
from jax import lax
from jax.experimental import pallas as pl
from jax.experimental.pallas import tpu as pltpu
from jax.experimental.pallas import tpu_sc as plsc

_NC = 2
_NS = 16
_NW = _NC * _NS
_LANES = 16

_TC_TILE = 1024
_NCHUNKS = 2


def _argmax_body(trt_ref, keyst_ref, idx_ref):
    tr_cm = trt_ref[0]
    keys_cm = keyst_ref[...]
    s = lax.dot_general(keys_cm, tr_cm, (((0,), (0,)), ((), ())),
                        preferred_element_type=jnp.float32)
    idx_ref[0, 0, :] = jnp.argmax(s, axis=0).astype(jnp.int32)


def _tc_argmax(trt3, keys_t, row_off, n_rows):
    b, c, l = trt3.shape
    m = keys_t.shape[1]
    jpr = l // _TC_TILE
    chunk_t = n_rows * l
    idx3 = pl.pallas_call(
        _argmax_body,
        grid=(n_rows, jpr),
        in_specs=[
            pl.BlockSpec((1, c, _TC_TILE), lambda bb, j: (bb + row_off, 0, j)),
            pl.BlockSpec((c, m), lambda bb, j: (0, 0)),
        ],
        out_specs=pl.BlockSpec((1, 1, _TC_TILE),
                               lambda bb, j: (bb * jpr + j, 0, 0)),
        out_shape=jax.ShapeDtypeStruct((n_rows * jpr, 1, _TC_TILE), jnp.int32),
        compiler_params=pltpu.CompilerParams(
            dimension_semantics=("arbitrary", "arbitrary"),
            fuse_transposed_lhs_in_matmul=True),
    )(trt3, keys_t)
    return idx3.reshape(chunk_t)


def _make_sc_mse(b, l, c, m, chunk_t):
    per_w = chunk_t // _NW
    n_groups = per_w // _LANES
    wpr = l // per_w
    mesh = plsc.VectorSubcoreMesh(core_axis_name="c", subcore_axis_name="s",
                                  num_cores=_NC, num_subcores=_NS)

    def make(row_off):
        @functools.partial(
            pl.kernel,
            out_type=[jax.ShapeDtypeStruct((chunk_t,), jnp.float32),
                      jax.ShapeDtypeStruct((chunk_t,), jnp.float32)],
            mesh=mesh,
            scratch_types=[
                pltpu.VMEM((c, m), jnp.float32),
                pltpu.VMEM((c, m), jnp.float32),
                pltpu.VMEM((c, per_w), jnp.float32),
                pltpu.VMEM((c, per_w), jnp.float32),
                pltpu.VMEM((per_w,), jnp.int32),
                pltpu.VMEM((per_w,), jnp.float32),
                pltpu.VMEM((per_w,), jnp.float32),
                pltpu.SemaphoreType.DMA,
                pltpu.SemaphoreType.DMA,
                pltpu.SemaphoreType.DMA,
                pltpu.SemaphoreType.DMA,
                pltpu.SemaphoreType.DMA,
            ],
            compiler_params=pltpu.CompilerParams(needs_layout_passes=False),
        )
        def sc_mse(trt_hbm, rept_hbm, keyst_hbm, valuest_hbm, idx_hbm,
                   outk_hbm, outv_hbm,
                   keys_v, values_v, tr_v, rep_v, idx_v, outk_v, outv_v,
                   sem0, sem1, sem2, sem3, sem4):
            wid = lax.axis_index("s") * _NC + lax.axis_index("c")
            base = wid * per_w
            bb = row_off + wid // wpr
            l0 = (wid % wpr) * per_w
            cps = [
                pltpu.async_copy(keyst_hbm, keys_v, sem0),
                pltpu.async_copy(valuest_hbm, values_v, sem1),
                pltpu.async_copy(trt_hbm.at[bb, :, pl.ds(l0, per_w)], tr_v,
                                 sem2),
                pltpu.async_copy(rept_hbm.at[bb, :, pl.ds(l0, per_w)], rep_v,
                                 sem3),
                pltpu.async_copy(idx_hbm.at[pl.ds(base, per_w)], idx_v, sem4),
            ]
            for cp in cps:
                cp.wait()
            lanes = lax.iota(jnp.int32, _LANES)

            def group_body(g, carry):
                rows = [g * 2 * _LANES + lanes, g * 2 * _LANES + _LANES + lanes]
                idxs = [plsc.load_gather(idx_v, [r]) for r in rows]
                acck = [[jnp.zeros((_LANES,), jnp.float32) for _ in range(2)]
                        for _ in range(2)]
                accv = [[jnp.zeros((_LANES,), jnp.float32) for _ in range(2)]
                        for _ in range(2)]
                for cc in range(c):
                    col = jnp.full((_LANES,), cc, jnp.int32)
                    par = cc & 1
                    for gi in range(2):
                        trc = plsc.load_gather(tr_v, [col, rows[gi]])
                        kc = plsc.load_gather(keys_v, [col, idxs[gi]])
                        dk = trc - kc
                        acck[gi][par] = acck[gi][par] + dk * dk
                        rc = plsc.load_gather(rep_v, [col, rows[gi]])
                        vc = plsc.load_gather(values_v, [col, idxs[gi]])
                        dv = rc - vc
                        accv[gi][par] = accv[gi][par] + dv * dv
                for gi in range(2):
                    plsc.store_scatter(outk_v, [rows[gi]],
                                       acck[gi][0] + acck[gi][1])
                    plsc.store_scatter(outv_v, [rows[gi]],
                                       accv[gi][0] + accv[gi][1])
                return carry

            lax.fori_loop(0, n_groups // 2, group_body, 0)
            pltpu.sync_copy(outk_v, outk_hbm.at[pl.ds(base, per_w)])
            pltpu.sync_copy(outv_v, outv_hbm.at[pl.ds(base, per_w)])

        return sc_mse

    return make


def kernel(trend_representation, representation, keys, values):
    b, l, c = trend_representation.shape
    m = keys.shape[0]
    trt3 = jnp.transpose(trend_representation, (0, 2, 1))
    rept3 = jnp.transpose(representation, (0, 2, 1))
    keys_t = keys.T
    values_t = values.T
    rows_per_chunk = b // _NCHUNKS
    chunk_t = rows_per_chunk * l
    sc_make = _make_sc_mse(b, l, c, m, chunk_t)
    kgs, vgs = [], []
    for i in range(_NCHUNKS):
        row_off = i * rows_per_chunk
        idx = _tc_argmax(trt3, keys_t, row_off, rows_per_chunk)
        kg, vg = sc_make(row_off)(trt3, rept3, keys_t, values_t, idx)
        kgs.append(kg)
        vgs.append(vg)
    kg = jnp.concatenate(kgs).reshape(b, l)
    vg = jnp.concatenate(vgs).reshape(b, l)
    return kg, vg


# speedup vs baseline: 1.5709x; 1.0447x over previous
import functools

import jax
import jax.numpy as jnp
from jax import lax
from jax.experimental import pallas as pl
from jax.experimental.pallas import tpu as pltpu
from jax.experimental.pallas import tpu_sc as plsc

_NC = 2
_NS = 16
_NW = _NC * _NS
_LANES = 16

_TC_TILE = 1024
_NCHUNKS = 1


def _argmax_body(trt_ref, keyst_ref, idx_ref):
    tr_cm = trt_ref[0]
    keys_cm = keyst_ref[...]
    s = lax.dot_general(keys_cm, tr_cm, (((0,), (0,)), ((), ())),
                        preferred_element_type=jnp.float32)
    idx_ref[0, 0, :] = jnp.argmax(s, axis=0).astype(jnp.int32)


def _tc_argmax(trt3, keys_t, row_off, n_rows):
    b, c, l = trt3.shape
    m = keys_t.shape[1]
    jpr = l // _TC_TILE
    chunk_t = n_rows * l
    idx3 = pl.pallas_call(
        _argmax_body,
        grid=(n_rows, jpr),
        in_specs=[
            pl.BlockSpec((1, c, _TC_TILE), lambda bb, j: (bb + row_off, 0, j)),
            pl.BlockSpec((c, m), lambda bb, j: (0, 0)),
        ],
        out_specs=pl.BlockSpec((1, 1, _TC_TILE),
                               lambda bb, j: (bb * jpr + j, 0, 0)),
        out_shape=jax.ShapeDtypeStruct((n_rows * jpr, 1, _TC_TILE), jnp.int32),
        compiler_params=pltpu.CompilerParams(
            dimension_semantics=("arbitrary", "arbitrary"),
            fuse_transposed_lhs_in_matmul=True),
    )(trt3, keys_t)
    return idx3.reshape(chunk_t)


def _make_sc_mse(b, l, c, m, chunk_t):
    per_w = chunk_t // _NW
    n_groups = per_w // _LANES
    wpr = l // per_w
    mesh = plsc.VectorSubcoreMesh(core_axis_name="c", subcore_axis_name="s",
                                  num_cores=_NC, num_subcores=_NS)

    def make(row_off):
        @functools.partial(
            pl.kernel,
            out_type=[jax.ShapeDtypeStruct((chunk_t,), jnp.float32),
                      jax.ShapeDtypeStruct((chunk_t,), jnp.float32)],
            mesh=mesh,
            scratch_types=[
                pltpu.VMEM((c, m), jnp.float32),
                pltpu.VMEM((c, m), jnp.float32),
                pltpu.VMEM((c, 512), jnp.float32),
                pltpu.VMEM((c, 512), jnp.float32),
                pltpu.VMEM((per_w,), jnp.int32),
                pltpu.VMEM((per_w,), jnp.float32),
                pltpu.VMEM((per_w,), jnp.float32),
                pltpu.SemaphoreType.DMA,
                pltpu.SemaphoreType.DMA,
                pltpu.SemaphoreType.DMA,
                pltpu.SemaphoreType.DMA,
                pltpu.SemaphoreType.DMA,
            ],
            compiler_params=pltpu.CompilerParams(needs_layout_passes=False),
        )
        def sc_mse(trt_hbm, rept_hbm, keyst_hbm, valuest_hbm, idx_hbm,
                   outk_hbm, outv_hbm,
                   keys_v, values_v, tr_v, rep_v, idx_v, outk_v, outv_v,
                   sem0, sem1, sem2, sem3, sem4):
            wid = lax.axis_index("s") * _NC + lax.axis_index("c")
            base = wid * per_w
            bb = row_off + wid // wpr
            l0 = (wid % wpr) * per_w
            sub = min(per_w, 512)
            nsub = per_w // sub
            cps = [
                pltpu.async_copy(keyst_hbm, keys_v, sem0),
                pltpu.async_copy(valuest_hbm, values_v, sem1),
                pltpu.async_copy(idx_hbm.at[pl.ds(base, per_w)], idx_v, sem4),
            ]
            for cp in cps:
                cp.wait()
            lanes = lax.iota(jnp.int32, _LANES)

            for ci in range(nsub):
                tr_cp = pltpu.async_copy(
                    trt_hbm.at[bb, :, pl.ds(l0 + ci * sub, sub)], tr_v, sem2)
                rep_cp = pltpu.async_copy(
                    rept_hbm.at[bb, :, pl.ds(l0 + ci * sub, sub)], rep_v, sem3)
                tr_cp.wait()
                rep_cp.wait()

                def group_body(g, carry, ci=ci):
                    loc = [g * 2 * _LANES + lanes,
                           g * 2 * _LANES + _LANES + lanes]
                    rows = [r + ci * sub for r in loc]
                    idxs = [plsc.load_gather(idx_v, [r]) for r in rows]
                    acck = [[jnp.zeros((_LANES,), jnp.float32)
                             for _ in range(2)] for _ in range(2)]
                    accv = [[jnp.zeros((_LANES,), jnp.float32)
                             for _ in range(2)] for _ in range(2)]
                    for cc in range(c):
                        col = jnp.full((_LANES,), cc, jnp.int32)
                        par = cc & 1
                        for gi in range(2):
                            trc = plsc.load_gather(tr_v, [col, loc[gi]])
                            kc = plsc.load_gather(keys_v, [col, idxs[gi]])
                            dk = trc - kc
                            acck[gi][par] = acck[gi][par] + dk * dk
                            rc = plsc.load_gather(rep_v, [col, loc[gi]])
                            vc = plsc.load_gather(values_v, [col, idxs[gi]])
                            dv = rc - vc
                            accv[gi][par] = accv[gi][par] + dv * dv
                    for gi in range(2):
                        plsc.store_scatter(outk_v, [rows[gi]],
                                           acck[gi][0] + acck[gi][1])
                        plsc.store_scatter(outv_v, [rows[gi]],
                                           accv[gi][0] + accv[gi][1])
                    return carry

                lax.fori_loop(0, sub // (2 * _LANES), group_body, 0)

            pltpu.sync_copy(outk_v, outk_hbm.at[pl.ds(base, per_w)])
            pltpu.sync_copy(outv_v, outv_hbm.at[pl.ds(base, per_w)])

        return sc_mse

    return make


def kernel(trend_representation, representation, keys, values):
    b, l, c = trend_representation.shape
    m = keys.shape[0]
    trt3 = jnp.transpose(trend_representation, (0, 2, 1))
    rept3 = jnp.transpose(representation, (0, 2, 1))
    keys_t = keys.T
    values_t = values.T
    rows_per_chunk = b // _NCHUNKS
    chunk_t = rows_per_chunk * l
    sc_make = _make_sc_mse(b, l, c, m, chunk_t)
    kgs, vgs = [], []
    for i in range(_NCHUNKS):
        row_off = i * rows_per_chunk
        idx = _tc_argmax(trt3, keys_t, row_off, rows_per_chunk)
        kg, vg = sc_make(row_off)(trt3, rept3, keys_t, values_t, idx)
        kgs.append(kg)
        vgs.append(vg)
    kg = jnp.concatenate(kgs).reshape(b, l)
    vg = jnp.concatenate(vgs).reshape(b, l)
    return kg, vg


# speedup vs baseline: 1.6468x; 1.0483x over previous
import functools

import jax
import jax.numpy as jnp
from jax import lax
from jax.experimental import pallas as pl
from jax.experimental.pallas import tpu as pltpu
from jax.experimental.pallas import tpu_sc as plsc

_NC = 2
_NS = 16
_NW = _NC * _NS
_LANES = 16

_TC_TILE = 1024
_NCHUNKS = 1


def _argmax_body(trt_ref, keyst_ref, idx_ref):
    tr_cm = trt_ref[0]
    keys_cm = keyst_ref[...]
    s = lax.dot_general(keys_cm, tr_cm, (((0,), (0,)), ((), ())),
                        preferred_element_type=jnp.float32)
    idx_ref[0, 0, :] = jnp.argmax(s, axis=0).astype(jnp.int32)


def _tc_argmax(trt3, keys_t, row_off, n_rows):
    b, c, l = trt3.shape
    m = keys_t.shape[1]
    jpr = l // _TC_TILE
    chunk_t = n_rows * l
    idx3 = pl.pallas_call(
        _argmax_body,
        grid=(n_rows, jpr),
        in_specs=[
            pl.BlockSpec((1, c, _TC_TILE), lambda bb, j: (bb + row_off, 0, j)),
            pl.BlockSpec((c, m), lambda bb, j: (0, 0)),
        ],
        out_specs=pl.BlockSpec((1, 1, _TC_TILE),
                               lambda bb, j: (bb * jpr + j, 0, 0)),
        out_shape=jax.ShapeDtypeStruct((n_rows * jpr, 1, _TC_TILE), jnp.int32),
        compiler_params=pltpu.CompilerParams(
            dimension_semantics=("arbitrary", "arbitrary"),
            fuse_transposed_lhs_in_matmul=True),
    )(trt3, keys_t)
    return idx3.reshape(chunk_t)


def _make_sc_mse(b, l, c, m, chunk_t):
    per_w = chunk_t // _NW
    n_groups = per_w // _LANES
    wpr = l // per_w
    mesh = plsc.VectorSubcoreMesh(core_axis_name="c", subcore_axis_name="s",
                                  num_cores=_NC, num_subcores=_NS)

    def make(row_off):
        @functools.partial(
            pl.kernel,
            out_type=[jax.ShapeDtypeStruct((b, l), jnp.float32),
                      jax.ShapeDtypeStruct((b, l), jnp.float32)],
            mesh=mesh,
            scratch_types=[
                pltpu.VMEM((c, m), jnp.float32),
                pltpu.VMEM((c, m), jnp.float32),
                pltpu.VMEM((c, 512), jnp.float32),
                pltpu.VMEM((c, 512), jnp.float32),
                pltpu.VMEM((per_w,), jnp.int32),
                pltpu.VMEM((per_w,), jnp.float32),
                pltpu.VMEM((per_w,), jnp.float32),
                pltpu.SemaphoreType.DMA,
                pltpu.SemaphoreType.DMA,
                pltpu.SemaphoreType.DMA,
                pltpu.SemaphoreType.DMA,
                pltpu.SemaphoreType.DMA,
            ],
            compiler_params=pltpu.CompilerParams(needs_layout_passes=False),
        )
        def sc_mse(trt_hbm, rept_hbm, keyst_hbm, valuest_hbm, idx_hbm,
                   outk_hbm, outv_hbm,
                   keys_v, values_v, tr_v, rep_v, idx_v, outk_v, outv_v,
                   sem0, sem1, sem2, sem3, sem4):
            wid = lax.axis_index("s") * _NC + lax.axis_index("c")
            base = wid * per_w
            bb = row_off + wid // wpr
            l0 = (wid % wpr) * per_w
            sub = min(per_w, 512)
            nsub = per_w // sub
            cps = [
                pltpu.async_copy(keyst_hbm, keys_v, sem0),
                pltpu.async_copy(valuest_hbm, values_v, sem1),
                pltpu.async_copy(idx_hbm.at[pl.ds(base, per_w)], idx_v, sem4),
            ]
            for cp in cps:
                cp.wait()
            lanes = lax.iota(jnp.int32, _LANES)

            for ci in range(nsub):
                tr_cp = pltpu.async_copy(
                    trt_hbm.at[bb, :, pl.ds(l0 + ci * sub, sub)], tr_v, sem2)
                rep_cp = pltpu.async_copy(
                    rept_hbm.at[bb, :, pl.ds(l0 + ci * sub, sub)], rep_v, sem3)
                tr_cp.wait()
                rep_cp.wait()

                def group_body(g, carry, ci=ci):
                    loc = [g * 2 * _LANES + lanes,
                           g * 2 * _LANES + _LANES + lanes]
                    rows = [r + ci * sub for r in loc]
                    idxs = [plsc.load_gather(idx_v, [r]) for r in rows]
                    acck = [[jnp.zeros((_LANES,), jnp.float32)
                             for _ in range(2)] for _ in range(2)]
                    accv = [[jnp.zeros((_LANES,), jnp.float32)
                             for _ in range(2)] for _ in range(2)]
                    for cc in range(c):
                        col = jnp.full((_LANES,), cc, jnp.int32)
                        par = cc & 1
                        for gi in range(2):
                            trc = plsc.load_gather(tr_v, [col, loc[gi]])
                            kc = plsc.load_gather(keys_v, [col, idxs[gi]])
                            dk = trc - kc
                            acck[gi][par] = acck[gi][par] + dk * dk
                            rc = plsc.load_gather(rep_v, [col, loc[gi]])
                            vc = plsc.load_gather(values_v, [col, idxs[gi]])
                            dv = rc - vc
                            accv[gi][par] = accv[gi][par] + dv * dv
                    for gi in range(2):
                        plsc.store_scatter(outk_v, [rows[gi]],
                                           acck[gi][0] + acck[gi][1])
                        plsc.store_scatter(outv_v, [rows[gi]],
                                           accv[gi][0] + accv[gi][1])
                    return carry

                lax.fori_loop(0, sub // (2 * _LANES), group_body, 0)

            pltpu.sync_copy(outk_v, outk_hbm.at[bb, pl.ds(l0, per_w)])
            pltpu.sync_copy(outv_v, outv_hbm.at[bb, pl.ds(l0, per_w)])

        return sc_mse

    return make


def kernel(trend_representation, representation, keys, values):
    b, l, c = trend_representation.shape
    m = keys.shape[0]
    trt3 = jnp.transpose(trend_representation, (0, 2, 1))
    rept3 = jnp.transpose(representation, (0, 2, 1))
    keys_t = keys.T
    values_t = values.T
    rows_per_chunk = b // _NCHUNKS
    chunk_t = rows_per_chunk * l
    sc_make = _make_sc_mse(b, l, c, m, chunk_t)
    idx = _tc_argmax(trt3, keys_t, 0, rows_per_chunk)
    kg, vg = sc_make(0)(trt3, rept3, keys_t, values_t, idx)
    return kg, vg
